# baseline (speedup 1.0000x reference)
import functools

import jax
import jax.numpy as jnp
from jax import lax
from jax.experimental import pallas as pl
from jax.experimental.pallas import tpu as pltpu
from jax.experimental.pallas import tpu_sc as plsc

_N = 10000
_E = 320000
_D = 128

_NC = 2
_NS = 16
_CH = 128
_GRP = 8
_NPAIR = 5
_NGRP = _NPAIR * 2
_CPT = _NGRP * _GRP
_CPA = _CPT + _GRP
_EPT = _CH * _CPT
_NT = _NC * _NS
_E_PAD = _EPT * _NT
_NPAD = 10240
_RP = _NPAD // _NS


def _mm_body(x_ref, w_ref, o_ref):
    o_ref[...] = jnp.dot(x_ref[...], w_ref[...],
                         preferred_element_type=jnp.float32)


def _matmul(x, W):
    return pl.pallas_call(
        _mm_body,
        grid=(10,),
        in_specs=[
            pl.BlockSpec((1000, _D), lambda r: (r, 0)),
            pl.BlockSpec((_D, _D), lambda r: (0, 0)),
        ],
        out_specs=pl.BlockSpec((1000, _D), lambda r: (r, 0)),
        out_shape=jax.ShapeDtypeStruct((_N, _D), jnp.float32),
    )(x, W)


def _add_body(a_ref, b_ref, o_ref):
    o_ref[...] = a_ref[0] + b_ref[0]


def _combine(parts):
    return pl.pallas_call(
        _add_body,
        grid=(10,),
        in_specs=[
            pl.BlockSpec((1, 1000, _D), lambda r: (0, r, 0)),
            pl.BlockSpec((1, 1000, _D), lambda r: (1, r, 0)),
        ],
        out_specs=pl.BlockSpec((1000, _D), lambda r: (r, 0)),
        out_shape=jax.ShapeDtypeStruct((_N, _D), jnp.float32),
    )(parts, parts)


@functools.partial(
    pl.kernel,
    out_type=jax.ShapeDtypeStruct((_NC, _NPAD, _D), jnp.float32),
    mesh=plsc.VectorSubcoreMesh(core_axis_name="c", subcore_axis_name="s"),
    scratch_types=[
        pltpu.VMEM((2, _GRP, _CH), jnp.int32),
        pltpu.VMEM((2, _GRP, _CH), jnp.int32),
        pltpu.VMEM((2, _GRP, _CH), jnp.float32),
        pltpu.VMEM((_CH, _D), jnp.float32),
        pltpu.VMEM((_CH, _D), jnp.float32),
        pltpu.VMEM_SHARED((_NPAD, _D), jnp.float32),
        pltpu.SemaphoreType.DMA,
        pltpu.SemaphoreType.DMA,
        pltpu.SemaphoreType.DMA,
        pltpu.SemaphoreType.DMA,
        pltpu.SemaphoreType.DMA,
        pltpu.SemaphoreType.DMA,
        pltpu.SemaphoreType.DMA,
        pltpu.SemaphoreType.DMA,
        pltpu.SemaphoreType.DMA,
        pltpu.SemaphoreType.DMA,
    ],
)
def _sc_agg(h_hbm, cols_hbm, rows_hbm, vals_hbm, out_hbm,
            colb, rowb, valb, rb0, rb1, acc,
            gs0, gs1, ss0, ss1, ic0, ic1, ir0, ir1, iv0, iv1):
    c = lax.axis_index("c")
    s = lax.axis_index("s")
    rbufs = (rb0, rb1)
    gsems = (gs0, gs1)
    ssems = (ss0, ss1)
    icsems = (ic0, ic1)
    irsems = (ir0, ir1)
    ivsems = (iv0, iv1)
    tid = c * _NS + s

    def fetch_idx(j, slot):
        pltpu.async_copy(cols_hbm.at[tid, pl.ds(j * _GRP, _GRP)],
                         colb.at[slot], icsems[slot])
        pltpu.async_copy(rows_hbm.at[tid, pl.ds(j * _GRP, _GRP)],
                         rowb.at[slot], irsems[slot])
        pltpu.async_copy(vals_hbm.at[tid, pl.ds(j * _GRP, _GRP)],
                         valb.at[slot], ivsems[slot])

    def wait_idx(slot):
        pltpu.make_async_copy(cols_hbm.at[0, pl.ds(0, _GRP)],
                              colb.at[slot], icsems[slot]).wait()
        pltpu.make_async_copy(rows_hbm.at[0, pl.ds(0, _GRP)],
                              rowb.at[slot], irsems[slot]).wait()
        pltpu.make_async_copy(vals_hbm.at[0, pl.ds(0, _GRP)],
                              valb.at[slot], ivsems[slot]).wait()

    def issue_gather(slot, k, b):
        pltpu.async_copy(h_hbm.at[colb.at[slot, k]], rbufs[b], gsems[b])

    def wait_gather(b):
        pltpu.make_async_copy(h_hbm.at[colb.at[0, 0]], rbufs[b],
                              gsems[b]).wait()

    def issue_scatter(slot, k, b):
        pltpu.async_copy(rbufs[b], acc.at[rowb.at[slot, k]], ssems[b],
                         add=True)

    def wait_scatter(b):
        pltpu.make_async_copy(rbufs[b], acc.at[rowb.at[0, 0]],
                              ssems[b]).wait()

    def compute(slot, k, b):
        rb = rbufs[b]

        def _edge16(g16, carry):
            vv = valb[slot, k, pl.ds(g16 * 16, 16)]
            for i in range(16):
                e = g16 * 16 + i
                sp = vv[i]
                for j in range(_D // 16):
                    rb[e, pl.ds(j * 16, 16)] = rb[e, pl.ds(j * 16, 16)] * sp
            return carry
        lax.fori_loop(0, _CH // 16, _edge16, 0)

    fetch_idx(0, 0)

    def _zrow(r, carry):
        for j in range(_D // 16):
            rb0[r, pl.ds(j * 16, 16)] = jnp.zeros((16,), jnp.float32)
        return carry
    lax.fori_loop(0, _CH, _zrow, 0)
    for k in range(_RP // _CH):
        pltpu.sync_copy(rb0, acc.at[pl.ds(s * _RP + k * _CH, _CH), :])

    wait_idx(0)
    issue_gather(0, 0, 0)
    issue_gather(0, 1, 1)
    plsc.subcore_barrier()

    def step(jj, t):
        slot = t // 8
        b = t % 2
        k = t % 8
        wait_gather(b)
        compute(slot, k, b)
        issue_scatter(slot, k, b)
        wait_scatter(b)
        if t % 8 == 0:
            fetch_idx(2 * jj + t // 8 + 1, (t // 8 + 1) % 2)
        k2 = (t + 2) % 8
        slot2 = ((t + 2) // 8) % 2
        if k2 == 0:
            wait_idx(slot2)
        issue_gather(slot2, k2, b)

    def _pair(jj, carry):
        for t in range(16):
            step(jj, t)
        return carry
    lax.fori_loop(0, _NPAIR, _pair, 0)
    wait_gather(0)
    wait_gather(1)

    plsc.subcore_barrier()
    pltpu.sync_copy(acc.at[pl.ds(s * _RP, _RP), :],
                    out_hbm.at[c, pl.ds(s * _RP, _RP), :])


def kernel(x, edge_index, adj_values, W):
    ei = edge_index.astype(jnp.int32)
    pad = _E_PAD - _E

    def _tile(arr):
        core = jnp.pad(arr, (0, pad)).reshape(_NT, _CPT, _CH)
        slack = jnp.zeros((_NT, _GRP, _CH), arr.dtype)
        return jnp.concatenate([core, slack], axis=1)

    rows_p = _tile(ei[0])
    cols_p = _tile(ei[1])
    vals_p = _tile(adj_values)
    h = _matmul(x, W)
    parts = _sc_agg(h, cols_p, rows_p, vals_p)
    return _combine(parts)
